# speedup vs baseline: 3.9525x; 3.9525x over previous
import functools

import jax
import jax.numpy as jnp
import numpy as np
from jax import lax
from jax.experimental import pallas as pl
from jax.experimental.pallas import tpu as pltpu
from jax.experimental.pallas import tpu_sc as plsc

_B, _S, _D, _V = 1024, 200, 128, 100000
_N = _B * _S
_NC, _NS = 2, 16
_NW = _NC * _NS
_RPW = _N // _NW
_CHUNKS = _RPW // _S
_LANES = 16


def _pos_table():
    i = np.arange(_D)[np.newaxis, :]
    embeds = 1.0 / np.power(10000.0, 2 * (i // 2) / np.float32(_D))
    loc = np.arange(_S)[:, np.newaxis]
    pos = embeds * loc
    pos[:, ::2] = np.sin(pos[:, ::2])
    pos[:, 1::2] = np.cos(pos[:, 1::2])
    return jnp.asarray(pos, dtype=jnp.float32)


@functools.partial(
    pl.kernel,
    out_type=jax.ShapeDtypeStruct((_N, _D), jnp.float32),
    mesh=plsc.VectorSubcoreMesh(core_axis_name="c", subcore_axis_name="s"),
    scratch_types=[
        pltpu.VMEM((_S,), jnp.int32),
        pltpu.VMEM((_S, _D), jnp.float32),
        pltpu.VMEM((_S, _D), jnp.float32),
        pltpu.SemaphoreType.DMA,
    ],
)
def _sc_embed(idx_hbm, table_hbm, pos_hbm, out_hbm, idx_v, rows_v, pos_v, sem):
    wid = lax.axis_index("s") * _NC + lax.axis_index("c")
    base = wid * _RPW
    pltpu.sync_copy(pos_hbm, pos_v)

    def chunk_body(g, carry):
        off = base + g * _S
        pltpu.sync_copy(idx_hbm.at[pl.ds(off, _S)], idx_v)
        cp0 = pltpu.async_copy(
            table_hbm.at[idx_v.at[pl.ds(0, 128)]], rows_v.at[pl.ds(0, 128), :], sem)
        cp1 = pltpu.async_copy(
            table_hbm.at[idx_v.at[pl.ds(128, _S - 128)]],
            rows_v.at[pl.ds(128, _S - 128), :], sem)
        cp0.wait()
        cp1.wait()

        def row_body(r, c2):
            for cc in range(_D // _LANES):
                sl = pl.ds(cc * _LANES, _LANES)
                plsc.addupdate(rows_v.at[r, sl], pos_v[r, sl])
            return c2

        lax.fori_loop(0, _S, row_body, 0)
        pltpu.sync_copy(rows_v, out_hbm.at[pl.ds(off, _S)])
        return carry

    lax.fori_loop(0, _CHUNKS, chunk_body, 0)


def kernel(data, table):
    pos = _pos_table()
    out = _sc_embed(data.reshape(_N), table, pos)
    return out.reshape(_B, _S, _D)


# speedup vs baseline: 6.8275x; 1.7274x over previous
import functools

import jax
import jax.numpy as jnp
import numpy as np
from jax import lax
from jax.experimental import pallas as pl
from jax.experimental.pallas import tpu as pltpu
from jax.experimental.pallas import tpu_sc as plsc

_B, _S, _D, _V = 1024, 200, 128, 100000
_N = _B * _S
_NC, _NS = 2, 16
_NW = _NC * _NS
_RPW = _N // _NW
_CHUNKS = _RPW // _S
_LANES = 16
_NBUF = 4


def _pos_table():
    i = np.arange(_D)[np.newaxis, :]
    embeds = 1.0 / np.power(10000.0, 2 * (i // 2) / np.float32(_D))
    loc = np.arange(_S)[:, np.newaxis]
    pos = embeds * loc
    pos[:, ::2] = np.sin(pos[:, ::2])
    pos[:, 1::2] = np.cos(pos[:, 1::2])
    return jnp.asarray(pos, dtype=jnp.float32)


@functools.partial(
    pl.kernel,
    out_type=jax.ShapeDtypeStruct((_N, _D), jnp.float32),
    mesh=plsc.VectorSubcoreMesh(core_axis_name="c", subcore_axis_name="s"),
    scratch_types=[
        [pltpu.VMEM((_S,), jnp.int32) for _ in range(_NBUF)],
        [pltpu.VMEM((_S, _D), jnp.float32) for _ in range(_NBUF)],
        pltpu.VMEM((_S, _D), jnp.float32),
        [pltpu.SemaphoreType.DMA for _ in range(_NBUF)],
        [pltpu.SemaphoreType.DMA for _ in range(_NBUF)],
    ],
)
def _sc_embed(idx_hbm, table_hbm, pos_hbm, out_hbm, idx_v, rows_v, pos_v,
              gsem, wsem):
    wid = lax.axis_index("s") * _NC + lax.axis_index("c")
    base = wid * _RPW
    pltpu.sync_copy(pos_hbm, pos_v)

    def start_gather(g, b):
        off = base + g * _S
        pltpu.sync_copy(idx_hbm.at[pl.ds(off, _S)], idx_v[b])
        pltpu.async_copy(table_hbm.at[idx_v[b].at[pl.ds(0, 128)]],
                         rows_v[b].at[pl.ds(0, 128), :], gsem[b])
        pltpu.async_copy(table_hbm.at[idx_v[b].at[pl.ds(128, _S - 128)]],
                         rows_v[b].at[pl.ds(128, _S - 128), :], gsem[b])

    def wait_gather(b):
        pltpu.make_async_copy(table_hbm.at[idx_v[b].at[pl.ds(0, 128)]],
                              rows_v[b].at[pl.ds(0, 128), :], gsem[b]).wait()
        pltpu.make_async_copy(table_hbm.at[idx_v[b].at[pl.ds(128, _S - 128)]],
                              rows_v[b].at[pl.ds(128, _S - 128), :],
                              gsem[b]).wait()

    def wait_write(g, b):
        off = base + g * _S
        pltpu.make_async_copy(rows_v[b], out_hbm.at[pl.ds(off, _S)],
                              wsem[b]).wait()

    for b in range(2):
        start_gather(b, b)

    def group_body(g4, carry):
        for b in range(_NBUF):
            g = g4 * _NBUF + b
            b2 = (b + 2) % _NBUF

            @pl.when(g >= 2)
            def _retire():
                wait_write(g - 2, b2)

            @pl.when(g + 2 < _CHUNKS)
            def _launch():
                start_gather(g + 2, b2)

            wait_gather(b)

            def row_body(r, c2):
                for rr in range(2):
                    for cc in range(_D // _LANES):
                        sl = pl.ds(cc * _LANES, _LANES)
                        plsc.addupdate(rows_v[b].at[2 * r + rr, sl],
                                       pos_v[2 * r + rr, sl])
                return c2

            lax.fori_loop(0, _S // 2, row_body, 0)
            off = base + g * _S
            pltpu.async_copy(rows_v[b], out_hbm.at[pl.ds(off, _S)], wsem[b])
        return carry

    lax.fori_loop(0, _CHUNKS // _NBUF, group_body, 0)

    for g in (_CHUNKS - 2, _CHUNKS - 1):
        wait_write(g, g % _NBUF)


def kernel(data, table):
    pos = _pos_table()
    out = _sc_embed(data.reshape(_N), table, pos)
    return out.reshape(_B, _S, _D)


# speedup vs baseline: 7.2698x; 1.0648x over previous
import functools

import jax
import jax.numpy as jnp
import numpy as np
from jax import lax
from jax.experimental import pallas as pl
from jax.experimental.pallas import tpu as pltpu
from jax.experimental.pallas import tpu_sc as plsc

_B, _S, _D, _V = 1024, 200, 128, 100000
_N = _B * _S
_NC, _NS = 2, 16
_NW = _NC * _NS
_RPW = _N // _NW
_CHUNKS = _RPW // _S
_LANES = 16
_NBUF = 4


def _pos_table():
    i = np.arange(_D)[np.newaxis, :]
    embeds = 1.0 / np.power(10000.0, 2 * (i // 2) / np.float32(_D))
    loc = np.arange(_S)[:, np.newaxis]
    pos = embeds * loc
    pos[:, ::2] = np.sin(pos[:, ::2])
    pos[:, 1::2] = np.cos(pos[:, 1::2])
    return jnp.asarray(pos, dtype=jnp.float32)


@functools.partial(
    pl.kernel,
    out_type=jax.ShapeDtypeStruct((_N, _D), jnp.float32),
    mesh=plsc.VectorSubcoreMesh(core_axis_name="c", subcore_axis_name="s"),
    scratch_types=[
        [pltpu.VMEM((_S,), jnp.int32) for _ in range(_NBUF)],
        [pltpu.VMEM((_S, _D), jnp.float32) for _ in range(_NBUF)],
        pltpu.VMEM((_S, _D), jnp.float32),
        [pltpu.SemaphoreType.DMA for _ in range(_NBUF)],
        [pltpu.SemaphoreType.DMA for _ in range(_NBUF)],
        [pltpu.SemaphoreType.DMA for _ in range(_NBUF)],
    ],
)
def _sc_embed(idx_hbm, table_hbm, pos_hbm, out_hbm, idx_v, rows_v, pos_v,
              gsem, wsem, isem):
    wid = lax.axis_index("s") * _NC + lax.axis_index("c")
    base = wid * _RPW
    pltpu.sync_copy(pos_hbm, pos_v)

    def start_idx(g, b):
        off = base + g * _S
        pltpu.async_copy(idx_hbm.at[pl.ds(off, _S)], idx_v[b], isem[b])

    def wait_idx(g, b):
        off = base + g * _S
        pltpu.make_async_copy(idx_hbm.at[pl.ds(off, _S)], idx_v[b],
                              isem[b]).wait()

    def start_gather(g, b):
        wait_idx(g, b)
        pltpu.async_copy(table_hbm.at[idx_v[b].at[pl.ds(0, 128)]],
                         rows_v[b].at[pl.ds(0, 128), :], gsem[b])
        pltpu.async_copy(table_hbm.at[idx_v[b].at[pl.ds(128, _S - 128)]],
                         rows_v[b].at[pl.ds(128, _S - 128), :], gsem[b])

    def wait_gather(b):
        pltpu.make_async_copy(table_hbm.at[idx_v[b].at[pl.ds(0, 128)]],
                              rows_v[b].at[pl.ds(0, 128), :], gsem[b]).wait()
        pltpu.make_async_copy(table_hbm.at[idx_v[b].at[pl.ds(128, _S - 128)]],
                              rows_v[b].at[pl.ds(128, _S - 128), :],
                              gsem[b]).wait()

    def wait_write(g, b):
        off = base + g * _S
        pltpu.make_async_copy(rows_v[b], out_hbm.at[pl.ds(off, _S)],
                              wsem[b]).wait()

    for g in range(3):
        start_idx(g, g)
    for g in range(2):
        start_gather(g, g)

    def group_body(g4, carry):
        for b in range(_NBUF):
            g = g4 * _NBUF + b
            b2 = (b + 2) % _NBUF
            b3 = (b + 3) % _NBUF

            @pl.when(g + 3 < _CHUNKS)
            def _prefetch():
                start_idx(g + 3, b3)

            @pl.when(g >= 2)
            def _retire():
                wait_write(g - 2, b2)

            @pl.when(g + 2 < _CHUNKS)
            def _launch():
                start_gather(g + 2, b2)

            wait_gather(b)

            def row_body(r, c2):
                for rr in range(2):
                    for cc in range(_D // _LANES):
                        sl = pl.ds(cc * _LANES, _LANES)
                        plsc.addupdate(rows_v[b].at[2 * r + rr, sl],
                                       pos_v[2 * r + rr, sl])
                return c2

            lax.fori_loop(0, _S // 2, row_body, 0)
            off = base + g * _S
            pltpu.async_copy(rows_v[b], out_hbm.at[pl.ds(off, _S)], wsem[b])
        return carry

    lax.fori_loop(0, _CHUNKS // _NBUF, group_body, 0)

    for g in (_CHUNKS - 2, _CHUNKS - 1):
        wait_write(g, g % _NBUF)


def kernel(data, table):
    pos = _pos_table()
    out = _sc_embed(data.reshape(_N), table, pos)
    return out.reshape(_B, _S, _D)
